# baseline (speedup 1.0000x reference)
import functools

import jax
import jax.numpy as jnp
from jax import lax
from jax.experimental import pallas as pl
from jax.experimental.pallas import tpu as pltpu

N_GRAPHS = 16
N_ATOMS = 64
ATOM_TYPES = 64
H = 256
TE = 128
NH = 8
LG = 3
LT = 3
R2 = 25.0
K = 32
DH = H // NH

GPB = 4
NR = GPB * N_ATOMS
NE = NR * K

_INTERPRET = False


def _silu(x):
    return x * jax.nn.sigmoid(x)


def _ln(x, g, b):
    m = x.mean(-1, keepdims=True)
    v = ((x - m) ** 2).mean(-1, keepdims=True)
    return (x - m) / jnp.sqrt(v + 1e-5) * g + b


def _mm(a, b):
    return jnp.dot(a, b, preferred_element_type=jnp.float32)


def _mm_t(a, b):
    return lax.dot_general(a, b, (((1,), (1,)), ((), ())),
                           preferred_element_type=jnp.float32)


def _body(names, *refs):
    r = dict(zip(names, refs[:-1]))
    out_ref = refs[-1]
    f32 = jnp.float32

    th = r['theta'][...].reshape(NR, ATOM_TYPES)
    cond = r['cond'][0, 0]
    th = jnp.where(cond > 0.0, jax.nn.softmax(th, axis=-1), th)
    x = _mm(th, r['node_in_Wt'][...]) + r['node_in_b'][...]

    pxs = r['pxs'][...]
    pys = r['pys'][...]
    pzs = r['pzs'][...]
    pxf = pxs.reshape(NR, 1)
    pyf = pys.reshape(NR, 1)
    pzf = pzs.reshape(NR, 1)
    pre_pe = (pxf * r['pe_w1c0'][...] + pyf * r['pe_w1c1'][...]
              + pzf * r['pe_w1c2'][...] + r['pe_b1'][...])
    pe = _mm(_silu(pre_pe), r['pe_w2t'][...]) + r['pe_b2'][...]

    si = r['si'][...].reshape(GPB, TE // 2)
    emb = jnp.concatenate([jnp.sin(si), jnp.cos(si)], axis=-1)
    e = _silu(_mm(emb, r['te_w1t'][...]) + r['te_b1'][...])
    e = _mm(e, r['te_w2t'][...]) + r['te_b2'][...]
    temb = _mm(e, r['tp_Wt'][...]) + r['tp_b'][...]

    x = ((x + pe).reshape(GPB, N_ATOMS, H) + temb[:, None, :]).reshape(NR, H)

    pxl = r['pxl'][...]
    pyl = r['pyl'][...]
    pzl = r['pzl'][...]
    dx = pxs - pxl
    dy = pys - pyl
    dz = pzs - pzl
    d2 = dx * dx + dy * dy + dz * dz

    ii = lax.broadcasted_iota(jnp.int32, (1, N_ATOMS, N_ATOMS), 1)
    jj = lax.broadcasted_iota(jnp.int32, (1, N_ATOMS, N_ATOMS), 2)
    valid = (d2 <= R2) & (ii != jj)

    d2k = d2[:, :, None, :]
    d2j = d2[:, :, :, None]
    jj4 = lax.broadcasted_iota(jnp.int32, (1, 1, N_ATOMS, 1), 2)
    kk4 = lax.broadcasted_iota(jnp.int32, (1, 1, 1, N_ATOMS), 3)
    better = (d2k < d2j) | ((d2k == d2j) & (kk4 < jj4))
    rank = jnp.sum((better & valid[:, :, None, :]).astype(f32), axis=3)
    sel = valid & (rank < float(K))
    mask2 = sel.astype(f32).reshape(NR, N_ATOMS)
    nval = jnp.sum(mask2, axis=1, keepdims=True)

    slot = _mm(mask2, r['ut'][...])
    sel2 = sel.reshape(NR, N_ATOMS)
    ss = lax.broadcasted_iota(jnp.int32, (1, K, 1), 1).astype(f32)
    oh3 = ((slot[:, None, :] == ss) & sel2[:, None, :]).astype(f32)
    maskc = (lax.broadcasted_iota(jnp.int32, (NR, K), 1).astype(f32)
             < nval).astype(f32)

    ed = jnp.where(d2 > 0.0, jnp.sqrt(jnp.where(d2 > 0.0, d2, 1.0)), 0.0)
    den = ed + 1e-8
    exm = (-dx / den).reshape(NR, N_ATOMS)
    eym = (-dy / den).reshape(NR, N_ATOMS)
    ezm = (-dz / den).reshape(NR, N_ATOMS)
    edm = ed.reshape(NR, N_ATOMS)
    edc = jnp.sum(oh3 * edm[:, None, :], axis=2)
    exc = jnp.sum(oh3 * exm[:, None, :], axis=2)
    eyc = jnp.sum(oh3 * eym[:, None, :], axis=2)
    ezc = jnp.sum(oh3 * ezm[:, None, :], axis=2)

    df1 = _silu(edc[:, :, None] * r['dist_w1r'][...] + r['dist_b1r'][...])
    df = _mm(df1.reshape(NE, H // 2), r['dist_w2t'][...])
    df = df + r['dist_b2'][...]
    rf1 = _silu(exc[:, :, None] * r['dir_w1c0'][...]
                + eyc[:, :, None] * r['dir_w1c1'][...]
                + ezc[:, :, None] * r['dir_w1c2'][...] + r['dir_b1r'][...])
    rf = _mm(rf1.reshape(NE, H // 2), r['dir_w2t'][...])
    rf = rf + r['dir_b2'][...]
    ef = _ln(jnp.concatenate([df, rf], axis=-1),
             r['en_g'][...], r['en_b'][...])
    c_all = _mm(ef, r['W1c_all'][...])

    gmb1 = r['g_msg_b1'][...]
    gmb2 = r['g_msg_b2'][...]
    gl1g = r['g_ln1_g'][...]
    gl1b = r['g_ln1_b'][...]
    gfb1 = r['g_ffn_b1'][...]
    gfb2 = r['g_ffn_b2'][...]
    gl2g = r['g_ln2_g'][...]
    gl2b = r['g_ln2_b'][...]
    maskc3 = maskc[:, :, None]
    for l in range(LG):
        a = _mm(x, r['W1a_t'][l]) + gmb1[l:l + 1]
        b = _mm(x, r['W1b_t'][l])
        bgs = []
        for g in range(GPB):
            ohg = oh3[g * N_ATOMS:(g + 1) * N_ATOMS].reshape(N_ATOMS * K,
                                                             N_ATOMS)
            bgs.append(_mm(ohg, b[g * N_ATOMS:(g + 1) * N_ATOMS]))
        bg = jnp.concatenate(bgs, axis=0).reshape(NR, K, H)
        c = c_all[:, l * H:(l + 1) * H].reshape(NR, K, H)
        pre = a[:, None, :] + bg + c
        s = jnp.sum(maskc3 * _silu(pre), axis=1)
        mi = _mm(s, r['g_msg_w2t'][l]) + nval * gmb2[l:l + 1]
        x = _ln(x + mi, gl1g[l:l + 1], gl1b[l:l + 1])
        ff = _silu(_mm(x, r['g_ffn_w1t'][l]) + gfb1[l:l + 1])
        ff = _mm(ff, r['g_ffn_w2t'][l]) + gfb2[l:l + 1]
        x = _ln(x + ff, gl2g[l:l + 1], gl2b[l:l + 1])

    tib = r['t_in_b'][...]
    tob = r['t_out_b'][...]
    tl1g = r['t_ln1_g'][...]
    tl1b = r['t_ln1_b'][...]
    tfb1 = r['t_ffn_b1'][...]
    tfb2 = r['t_ffn_b2'][...]
    tl2g = r['t_ln2_g'][...]
    tl2b = r['t_ln2_b'][...]
    inv_sqrt_dh = jnp.sqrt(jnp.float32(DH))
    for l in range(LT):
        qkv = _mm(x, r['t_in_wt'][l]) + tib[l:l + 1]
        gouts = []
        for g in range(GPB):
            rows = slice(g * N_ATOMS, (g + 1) * N_ATOMS)
            outs = []
            for hh in range(NH):
                qh = qkv[rows, hh * DH:(hh + 1) * DH]
                kh = qkv[rows, H + hh * DH:H + (hh + 1) * DH]
                vh = qkv[rows, 2 * H + hh * DH:2 * H + (hh + 1) * DH]
                att = jax.nn.softmax(_mm_t(qh, kh) / inv_sqrt_dh, axis=-1)
                outs.append(_mm(att, vh))
            gouts.append(jnp.concatenate(outs, axis=-1))
        o = jnp.concatenate(gouts, axis=0)
        o = _mm(o, r['t_out_wt'][l]) + tob[l:l + 1]
        x = _ln(x + o, tl1g[l:l + 1], tl1b[l:l + 1])
        hf = _mm(x, r['t_ffn_w1t'][l]) + tfb1[l:l + 1]
        hf = 0.5 * hf * (1.0 + lax.erf(hf / jnp.sqrt(jnp.float32(2.0))))
        ff = _mm(hf, r['t_ffn_w2t'][l]) + tfb2[l:l + 1]
        x = _ln(x + ff, tl2g[l:l + 1], tl2b[l:l + 1])

    x3 = x.reshape(GPB, N_ATOMS, H)
    mean_p = jnp.mean(x3, axis=1)
    max_p = jnp.max(x3, axis=1)
    gf = jnp.concatenate([mean_p, max_p], axis=-1)
    og = _silu(_mm(gf, r['pool_Wt'][...]) + r['pool_b'][...])
    out_ref[...] = og.reshape(1, GPB, H)


def _graph_spec(a):
    nz = a.ndim - 1
    return pl.BlockSpec((GPB,) + a.shape[1:],
                        lambda g, _nz=nz: (g,) + (0,) * _nz)


def _const_spec(a):
    nd = a.ndim
    return pl.BlockSpec(a.shape, lambda g, _nd=nd: (0,) * _nd)


def kernel(theta_t, pos_t, t, batch, params):
    p = params
    f32 = jnp.float32

    theta3 = theta_t.reshape(N_GRAPHS, N_ATOMS, ATOM_TYPES)
    pos3 = pos_t.reshape(N_GRAPHS, N_ATOMS, 3)
    pxs = pos3[..., 0:1]
    pys = pos3[..., 1:2]
    pzs = pos3[..., 2:3]
    pxl = jnp.swapaxes(pxs, 1, 2)
    pyl = jnp.swapaxes(pys, 1, 2)
    pzl = jnp.swapaxes(pzs, 1, 2)

    half = TE // 2
    inv_freq = 1.0 / (10000.0 ** (jnp.arange(half, dtype=f32) / half))
    si = (t[:, None] * inv_freq[None, :]).reshape(N_GRAPHS, 1, half)

    cond = ((theta_t.min() < 0) | (theta_t.max() > 1.0))
    condf = cond.astype(f32).reshape(1, 1)

    w1t = jnp.swapaxes(p['g_msg_w1'], 1, 2)

    graph_inputs = [
        ('theta', theta3),
        ('pxs', pxs), ('pys', pys), ('pzs', pzs),
        ('pxl', pxl), ('pyl', pyl), ('pzl', pzl),
        ('si', si),
    ]
    const_inputs = [
        ('cond', condf),
        ('node_in_Wt', p['node_in_W'].T),
        ('node_in_b', p['node_in_b'].reshape(1, H)),
        ('pe_w1c0', p['pe_w1'][:, 0].reshape(1, H)),
        ('pe_w1c1', p['pe_w1'][:, 1].reshape(1, H)),
        ('pe_w1c2', p['pe_w1'][:, 2].reshape(1, H)),
        ('pe_b1', p['pe_b1'].reshape(1, H)),
        ('pe_w2t', p['pe_w2'].T),
        ('pe_b2', p['pe_b2'].reshape(1, H)),
        ('te_w1t', p['te_w1'].T),
        ('te_b1', p['te_b1'].reshape(1, TE)),
        ('te_w2t', p['te_w2'].T),
        ('te_b2', p['te_b2'].reshape(1, TE)),
        ('tp_Wt', p['tp_W'].T),
        ('tp_b', p['tp_b'].reshape(1, H)),
        ('dist_w1r', p['dist_w1'].reshape(1, 1, H // 2)),
        ('dist_b1r', p['dist_b1'].reshape(1, 1, H // 2)),
        ('dist_w2t', p['dist_w2'].T),
        ('dist_b2', p['dist_b2'].reshape(1, H // 2)),
        ('dir_w1c0', p['dir_w1'][:, 0].reshape(1, 1, H // 2)),
        ('dir_w1c1', p['dir_w1'][:, 1].reshape(1, 1, H // 2)),
        ('dir_w1c2', p['dir_w1'][:, 2].reshape(1, 1, H // 2)),
        ('dir_b1r', p['dir_b1'].reshape(1, 1, H // 2)),
        ('dir_w2t', p['dir_w2'].T),
        ('dir_b2', p['dir_b2'].reshape(1, H // 2)),
        ('en_g', p['en_g'].reshape(1, H)),
        ('en_b', p['en_b'].reshape(1, H)),
        ('W1a_t', w1t[:, 0:H, :]),
        ('W1b_t', w1t[:, H:2 * H, :]),
        ('W1c_all', jnp.concatenate(
            [w1t[l, 2 * H:3 * H, :] for l in range(LG)], axis=1)),
        ('ut', jnp.triu(jnp.ones((N_ATOMS, N_ATOMS), f32), k=1)),
        ('g_msg_b1', p['g_msg_b1']),
        ('g_msg_w2t', jnp.swapaxes(p['g_msg_w2'], 1, 2)),
        ('g_msg_b2', p['g_msg_b2']),
        ('g_ln1_g', p['g_ln1_g']),
        ('g_ln1_b', p['g_ln1_b']),
        ('g_ffn_w1t', jnp.swapaxes(p['g_ffn_w1'], 1, 2)),
        ('g_ffn_b1', p['g_ffn_b1']),
        ('g_ffn_w2t', jnp.swapaxes(p['g_ffn_w2'], 1, 2)),
        ('g_ffn_b2', p['g_ffn_b2']),
        ('g_ln2_g', p['g_ln2_g']),
        ('g_ln2_b', p['g_ln2_b']),
        ('t_in_wt', jnp.swapaxes(p['t_in_w'], 1, 2)),
        ('t_in_b', p['t_in_b']),
        ('t_out_wt', jnp.swapaxes(p['t_out_w'], 1, 2)),
        ('t_out_b', p['t_out_b']),
        ('t_ln1_g', p['t_ln1_g']),
        ('t_ln1_b', p['t_ln1_b']),
        ('t_ffn_w1t', jnp.swapaxes(p['t_ffn_w1'], 1, 2)),
        ('t_ffn_b1', p['t_ffn_b1']),
        ('t_ffn_w2t', jnp.swapaxes(p['t_ffn_w2'], 1, 2)),
        ('t_ffn_b2', p['t_ffn_b2']),
        ('t_ln2_g', p['t_ln2_g']),
        ('t_ln2_b', p['t_ln2_b']),
        ('pool_Wt', p['pool_W'].T),
        ('pool_b', p['pool_b'].reshape(1, H)),
    ]

    names = [n for n, _ in graph_inputs] + [n for n, _ in const_inputs]
    arrays = [a for _, a in graph_inputs] + [a for _, a in const_inputs]
    in_specs = ([_graph_spec(a) for _, a in graph_inputs]
                + [_const_spec(a) for _, a in const_inputs])

    out = pl.pallas_call(
        functools.partial(_body, tuple(names)),
        grid=(N_GRAPHS // GPB,),
        in_specs=in_specs,
        out_specs=pl.BlockSpec((1, GPB, H), lambda g: (g, 0, 0)),
        out_shape=jax.ShapeDtypeStruct((N_GRAPHS // GPB, GPB, H), f32),
        compiler_params=pltpu.CompilerParams(
            dimension_semantics=("parallel",)),
        interpret=_INTERPRET,
    )(*arrays)
    return out.reshape(N_GRAPHS, H)


# speedup vs baseline: 1.1102x; 1.1102x over previous
import functools

import jax
import jax.numpy as jnp
from jax import lax
from jax.experimental import pallas as pl
from jax.experimental.pallas import tpu as pltpu

N_GRAPHS = 16
N_ATOMS = 64
ATOM_TYPES = 64
H = 256
TE = 128
NH = 8
LG = 3
LT = 3
R2 = 25.0
K = 32
DH = H // NH

GPB = 2
NR = GPB * N_ATOMS
NE = NR * K

_INTERPRET = False


def _silu(x):
    return x * jax.nn.sigmoid(x)


def _ln(x, g, b):
    m = x.mean(-1, keepdims=True)
    v = ((x - m) ** 2).mean(-1, keepdims=True)
    return (x - m) / jnp.sqrt(v + 1e-5) * g + b


def _mm(a, b):
    return jnp.dot(a, b, preferred_element_type=jnp.float32)


def _mm_t(a, b):
    return lax.dot_general(a, b, (((1,), (1,)), ((), ())),
                           preferred_element_type=jnp.float32)


def _body(names, *refs):
    r = dict(zip(names, refs[:-1]))
    out_ref = refs[-1]
    f32 = jnp.float32

    th = r['theta'][...].reshape(NR, ATOM_TYPES)
    cond = r['cond'][0, 0]
    th = jnp.where(cond > 0.0, jax.nn.softmax(th, axis=-1), th)
    x = _mm(th, r['node_in_Wt'][...]) + r['node_in_b'][...]

    pxs = r['pxs'][...]
    pys = r['pys'][...]
    pzs = r['pzs'][...]
    pxf = pxs.reshape(NR, 1)
    pyf = pys.reshape(NR, 1)
    pzf = pzs.reshape(NR, 1)
    pre_pe = (pxf * r['pe_w1c0'][...] + pyf * r['pe_w1c1'][...]
              + pzf * r['pe_w1c2'][...] + r['pe_b1'][...])
    pe = _mm(_silu(pre_pe), r['pe_w2t'][...]) + r['pe_b2'][...]

    si = r['si'][...].reshape(GPB, TE // 2)
    emb = jnp.concatenate([jnp.sin(si), jnp.cos(si)], axis=-1)
    e = _silu(_mm(emb, r['te_w1t'][...]) + r['te_b1'][...])
    e = _mm(e, r['te_w2t'][...]) + r['te_b2'][...]
    temb = _mm(e, r['tp_Wt'][...]) + r['tp_b'][...]

    x = ((x + pe).reshape(GPB, N_ATOMS, H) + temb[:, None, :]).reshape(NR, H)

    pxl = r['pxl'][...]
    pyl = r['pyl'][...]
    pzl = r['pzl'][...]
    dx = pxs - pxl
    dy = pys - pyl
    dz = pzs - pzl
    d2 = dx * dx + dy * dy + dz * dz

    ii = lax.broadcasted_iota(jnp.int32, (1, N_ATOMS, N_ATOMS), 1)
    jj = lax.broadcasted_iota(jnp.int32, (1, N_ATOMS, N_ATOMS), 2)
    valid = (d2 <= R2) & (ii != jj)

    d2k = d2[:, :, None, :]
    d2j = d2[:, :, :, None]
    jj4 = lax.broadcasted_iota(jnp.int32, (1, 1, N_ATOMS, 1), 2)
    kk4 = lax.broadcasted_iota(jnp.int32, (1, 1, 1, N_ATOMS), 3)
    better = (d2k < d2j) | ((d2k == d2j) & (kk4 < jj4))
    rank = jnp.sum((better & valid[:, :, None, :]).astype(f32), axis=3)
    sel = valid & (rank < float(K))
    mask2 = sel.astype(f32).reshape(NR, N_ATOMS)
    nval = jnp.sum(mask2, axis=1, keepdims=True)

    slot = _mm(mask2, r['ut'][...])
    sel2 = sel.reshape(NR, N_ATOMS)
    ss = lax.broadcasted_iota(jnp.int32, (1, K, 1), 1).astype(f32)
    oh3 = ((slot[:, None, :] == ss) & sel2[:, None, :]).astype(f32)
    maskc = (lax.broadcasted_iota(jnp.int32, (NR, K), 1).astype(f32)
             < nval).astype(f32)

    ed = jnp.where(d2 > 0.0, jnp.sqrt(jnp.where(d2 > 0.0, d2, 1.0)), 0.0)
    den = ed + 1e-8
    exm = (-dx / den).reshape(NR, N_ATOMS)
    eym = (-dy / den).reshape(NR, N_ATOMS)
    ezm = (-dz / den).reshape(NR, N_ATOMS)
    edm = ed.reshape(NR, N_ATOMS)
    edc = jnp.sum(oh3 * edm[:, None, :], axis=2)
    exc = jnp.sum(oh3 * exm[:, None, :], axis=2)
    eyc = jnp.sum(oh3 * eym[:, None, :], axis=2)
    ezc = jnp.sum(oh3 * ezm[:, None, :], axis=2)

    df1 = _silu(edc[:, :, None] * r['dist_w1r'][...] + r['dist_b1r'][...])
    df = _mm(df1.reshape(NE, H // 2), r['dist_w2t'][...])
    df = df + r['dist_b2'][...]
    rf1 = _silu(exc[:, :, None] * r['dir_w1c0'][...]
                + eyc[:, :, None] * r['dir_w1c1'][...]
                + ezc[:, :, None] * r['dir_w1c2'][...] + r['dir_b1r'][...])
    rf = _mm(rf1.reshape(NE, H // 2), r['dir_w2t'][...])
    rf = rf + r['dir_b2'][...]
    efc = jnp.concatenate([df, rf], axis=-1)
    m = efc.mean(-1, keepdims=True)
    v = ((efc - m) ** 2).mean(-1, keepdims=True)
    ef_n = (efc - m) / jnp.sqrt(v + 1e-5)
    c_all = (_mm(ef_n.astype(jnp.bfloat16), r['W1c_sg'][...])
             + r['W1c_bias'][...])

    gmb1 = r['g_msg_b1'][...]
    gmb2 = r['g_msg_b2'][...]
    gl1g = r['g_ln1_g'][...]
    gl1b = r['g_ln1_b'][...]
    gfb1 = r['g_ffn_b1'][...]
    gfb2 = r['g_ffn_b2'][...]
    gl2g = r['g_ln2_g'][...]
    gl2b = r['g_ln2_b'][...]
    maskc3 = maskc[:, :, None]
    for l in range(LG):
        a = _mm(x, r['W1a_t'][l]) + gmb1[l:l + 1]
        b = _mm(x, r['W1b_t'][l])
        bgs = []
        for g in range(GPB):
            ohg = oh3[g * N_ATOMS:(g + 1) * N_ATOMS].reshape(N_ATOMS * K,
                                                             N_ATOMS)
            bgs.append(_mm(ohg, b[g * N_ATOMS:(g + 1) * N_ATOMS]))
        bg = jnp.concatenate(bgs, axis=0).reshape(NR, K, H)
        c = c_all[:, l * H:(l + 1) * H].reshape(NR, K, H)
        pre = a[:, None, :] + bg + c
        s = jnp.sum(maskc3 * _silu(pre), axis=1)
        mi = _mm(s, r['g_msg_w2t'][l]) + nval * gmb2[l:l + 1]
        x = _ln(x + mi, gl1g[l:l + 1], gl1b[l:l + 1])
        ff = _silu(_mm(x, r['g_ffn_w1t'][l]) + gfb1[l:l + 1])
        ff = _mm(ff, r['g_ffn_w2t'][l]) + gfb2[l:l + 1]
        x = _ln(x + ff, gl2g[l:l + 1], gl2b[l:l + 1])

    tib = r['t_in_b'][...]
    tob = r['t_out_b'][...]
    tl1g = r['t_ln1_g'][...]
    tl1b = r['t_ln1_b'][...]
    tfb1 = r['t_ffn_b1'][...]
    tfb2 = r['t_ffn_b2'][...]
    tl2g = r['t_ln2_g'][...]
    tl2b = r['t_ln2_b'][...]
    inv_sqrt_dh = jnp.sqrt(jnp.float32(DH))
    for l in range(LT):
        qkv = _mm(x, r['t_in_wt'][l]) + tib[l:l + 1]
        gouts = []
        for g in range(GPB):
            rows = slice(g * N_ATOMS, (g + 1) * N_ATOMS)
            outs = []
            for hh in range(NH):
                qh = qkv[rows, hh * DH:(hh + 1) * DH]
                kh = qkv[rows, H + hh * DH:H + (hh + 1) * DH]
                vh = qkv[rows, 2 * H + hh * DH:2 * H + (hh + 1) * DH]
                att = jax.nn.softmax(_mm_t(qh, kh) / inv_sqrt_dh, axis=-1)
                outs.append(_mm(att, vh))
            gouts.append(jnp.concatenate(outs, axis=-1))
        o = jnp.concatenate(gouts, axis=0)
        o = _mm(o, r['t_out_wt'][l]) + tob[l:l + 1]
        x = _ln(x + o, tl1g[l:l + 1], tl1b[l:l + 1])
        hf = _mm(x, r['t_ffn_w1t'][l]) + tfb1[l:l + 1]
        hf = 0.5 * hf * (1.0 + lax.erf(hf / jnp.sqrt(jnp.float32(2.0))))
        ff = _mm(hf, r['t_ffn_w2t'][l]) + tfb2[l:l + 1]
        x = _ln(x + ff, tl2g[l:l + 1], tl2b[l:l + 1])

    x3 = x.reshape(GPB, N_ATOMS, H)
    mean_p = jnp.mean(x3, axis=1)
    max_p = jnp.max(x3, axis=1)
    gf = jnp.concatenate([mean_p, max_p], axis=-1)
    og = _silu(_mm(gf, r['pool_Wt'][...]) + r['pool_b'][...])
    out_ref[...] = og.reshape(1, GPB, H)


def _graph_spec(a):
    nz = a.ndim - 1
    return pl.BlockSpec((GPB,) + a.shape[1:],
                        lambda g, _nz=nz: (g,) + (0,) * _nz)


def _const_spec(a):
    nd = a.ndim
    return pl.BlockSpec(a.shape, lambda g, _nd=nd: (0,) * _nd)


def kernel(theta_t, pos_t, t, batch, params):
    p = params
    f32 = jnp.float32

    theta3 = theta_t.reshape(N_GRAPHS, N_ATOMS, ATOM_TYPES)
    pos3 = pos_t.reshape(N_GRAPHS, N_ATOMS, 3)
    pxs = pos3[..., 0:1]
    pys = pos3[..., 1:2]
    pzs = pos3[..., 2:3]
    pxl = jnp.swapaxes(pxs, 1, 2)
    pyl = jnp.swapaxes(pys, 1, 2)
    pzl = jnp.swapaxes(pzs, 1, 2)

    half = TE // 2
    inv_freq = 1.0 / (10000.0 ** (jnp.arange(half, dtype=f32) / half))
    si = (t[:, None] * inv_freq[None, :]).reshape(N_GRAPHS, 1, half)

    cond = ((theta_t.min() < 0) | (theta_t.max() > 1.0))
    condf = cond.astype(f32).reshape(1, 1)

    w1t = jnp.swapaxes(p['g_msg_w1'], 1, 2)

    graph_inputs = [
        ('theta', theta3),
        ('pxs', pxs), ('pys', pys), ('pzs', pzs),
        ('pxl', pxl), ('pyl', pyl), ('pzl', pzl),
        ('si', si),
    ]
    const_inputs = [
        ('cond', condf),
        ('node_in_Wt', p['node_in_W'].T),
        ('node_in_b', p['node_in_b'].reshape(1, H)),
        ('pe_w1c0', p['pe_w1'][:, 0].reshape(1, H)),
        ('pe_w1c1', p['pe_w1'][:, 1].reshape(1, H)),
        ('pe_w1c2', p['pe_w1'][:, 2].reshape(1, H)),
        ('pe_b1', p['pe_b1'].reshape(1, H)),
        ('pe_w2t', p['pe_w2'].T),
        ('pe_b2', p['pe_b2'].reshape(1, H)),
        ('te_w1t', p['te_w1'].T),
        ('te_b1', p['te_b1'].reshape(1, TE)),
        ('te_w2t', p['te_w2'].T),
        ('te_b2', p['te_b2'].reshape(1, TE)),
        ('tp_Wt', p['tp_W'].T),
        ('tp_b', p['tp_b'].reshape(1, H)),
        ('dist_w1r', p['dist_w1'].reshape(1, 1, H // 2)),
        ('dist_b1r', p['dist_b1'].reshape(1, 1, H // 2)),
        ('dist_w2t', p['dist_w2'].T),
        ('dist_b2', p['dist_b2'].reshape(1, H // 2)),
        ('dir_w1c0', p['dir_w1'][:, 0].reshape(1, 1, H // 2)),
        ('dir_w1c1', p['dir_w1'][:, 1].reshape(1, 1, H // 2)),
        ('dir_w1c2', p['dir_w1'][:, 2].reshape(1, 1, H // 2)),
        ('dir_b1r', p['dir_b1'].reshape(1, 1, H // 2)),
        ('dir_w2t', p['dir_w2'].T),
        ('dir_b2', p['dir_b2'].reshape(1, H // 2)),
        ('en_g', p['en_g'].reshape(1, H)),
        ('en_b', p['en_b'].reshape(1, H)),
        ('W1a_t', w1t[:, 0:H, :]),
        ('W1b_t', w1t[:, H:2 * H, :]),
        ('W1c_sg', (p['en_g'][:, None] * jnp.concatenate(
            [w1t[l, 2 * H:3 * H, :] for l in range(LG)],
            axis=1)).astype(jnp.bfloat16)),
        ('W1c_bias', (p['en_b'][None, :] @ jnp.concatenate(
            [w1t[l, 2 * H:3 * H, :] for l in range(LG)], axis=1))),
        ('ut', jnp.triu(jnp.ones((N_ATOMS, N_ATOMS), f32), k=1)),
        ('g_msg_b1', p['g_msg_b1']),
        ('g_msg_w2t', jnp.swapaxes(p['g_msg_w2'], 1, 2)),
        ('g_msg_b2', p['g_msg_b2']),
        ('g_ln1_g', p['g_ln1_g']),
        ('g_ln1_b', p['g_ln1_b']),
        ('g_ffn_w1t', jnp.swapaxes(p['g_ffn_w1'], 1, 2)),
        ('g_ffn_b1', p['g_ffn_b1']),
        ('g_ffn_w2t', jnp.swapaxes(p['g_ffn_w2'], 1, 2)),
        ('g_ffn_b2', p['g_ffn_b2']),
        ('g_ln2_g', p['g_ln2_g']),
        ('g_ln2_b', p['g_ln2_b']),
        ('t_in_wt', jnp.swapaxes(p['t_in_w'], 1, 2)),
        ('t_in_b', p['t_in_b']),
        ('t_out_wt', jnp.swapaxes(p['t_out_w'], 1, 2)),
        ('t_out_b', p['t_out_b']),
        ('t_ln1_g', p['t_ln1_g']),
        ('t_ln1_b', p['t_ln1_b']),
        ('t_ffn_w1t', jnp.swapaxes(p['t_ffn_w1'], 1, 2)),
        ('t_ffn_b1', p['t_ffn_b1']),
        ('t_ffn_w2t', jnp.swapaxes(p['t_ffn_w2'], 1, 2)),
        ('t_ffn_b2', p['t_ffn_b2']),
        ('t_ln2_g', p['t_ln2_g']),
        ('t_ln2_b', p['t_ln2_b']),
        ('pool_Wt', p['pool_W'].T),
        ('pool_b', p['pool_b'].reshape(1, H)),
    ]

    names = [n for n, _ in graph_inputs] + [n for n, _ in const_inputs]
    arrays = [a for _, a in graph_inputs] + [a for _, a in const_inputs]
    in_specs = ([_graph_spec(a) for _, a in graph_inputs]
                + [_const_spec(a) for _, a in const_inputs])

    out = pl.pallas_call(
        functools.partial(_body, tuple(names)),
        grid=(N_GRAPHS // GPB,),
        in_specs=in_specs,
        out_specs=pl.BlockSpec((1, GPB, H), lambda g: (g, 0, 0)),
        out_shape=jax.ShapeDtypeStruct((N_GRAPHS // GPB, GPB, H), f32),
        compiler_params=pltpu.CompilerParams(
            dimension_semantics=("parallel",)),
        interpret=_INTERPRET,
    )(*arrays)
    return out.reshape(N_GRAPHS, H)
